# baseline (speedup 1.0000x reference)
import functools

import jax
import jax.numpy as jnp
from jax import lax
from jax.experimental import pallas as pl
from jax.experimental.pallas import tpu as pltpu
from jax.experimental.pallas import tpu_sc as plsc

VOCAB = 1000000
D = 64
B = 16384
NC = 2
NS = 16
L = 16
NW = NC * NS
BPW = B // NW
CH = 128
NCH = BPW // CH
NG = CH // L

_mesh = plsc.VectorSubcoreMesh(core_axis_name="c", subcore_axis_name="s")


@functools.partial(
    pl.kernel,
    mesh=_mesh,
    compiler_params=pltpu.CompilerParams(
        needs_layout_passes=False, use_tc_tiling_on_sc=False),
    out_type=jax.ShapeDtypeStruct((B,), jnp.float32),
    scratch_types=[
        pltpu.VMEM((NCH, CH), jnp.int32),
        pltpu.VMEM((NCH, CH), jnp.int32),
        pltpu.VMEM((CH, D), jnp.float32),
        pltpu.VMEM((CH, D), jnp.float32),
        pltpu.VMEM((BPW,), jnp.float32),
        pltpu.VMEM((L, L), jnp.float32),
        pltpu.SemaphoreType.DMA,
    ],
)
def _sg_hs_kernel(emb, cs, nodes, out, cs_i, nd_i, a_v, b_v, o_v, t_v, sem):
    wid = lax.axis_index("s") * NC + lax.axis_index("c")
    base = wid * BPW

    for j in range(NCH):
        pltpu.sync_copy(cs.at[pl.ds(base + j * CH, CH)], cs_i.at[j])
        pltpu.sync_copy(nodes.at[pl.ds(base + j * CH, CH)], nd_i.at[j])

    for j in range(NCH):
        ga = pltpu.async_copy(emb.at[cs_i.at[j]], a_v, sem)
        gb = pltpu.async_copy(emb.at[nd_i.at[j]], b_v, sem)
        ga.wait()
        gb.wait()

        def group_body(g, carry):
            r0 = g * L
            for i in range(L):
                r = r0 + i
                partial = a_v[r, pl.ds(0, L)] * b_v[r, pl.ds(0, L)]
                for c in range(1, D // L):
                    partial = partial + (a_v[r, pl.ds(c * L, L)]
                                         * b_v[r, pl.ds(c * L, L)])
                t_v[i, pl.ds(0, L)] = partial
            rows16 = lax.iota(jnp.int32, L)
            y = plsc.load_gather(t_v, [rows16, jnp.zeros((L,), jnp.int32)])
            for col in range(1, L):
                y = y + plsc.load_gather(
                    t_v, [rows16, jnp.full((L,), col, jnp.int32)])
            y = 1.0 / (1.0 + jnp.exp(-y))
            o_v[pl.ds(j * CH + r0, L)] = y
            return carry

        lax.fori_loop(0, NG, group_body, 0)

    pltpu.sync_copy(o_v, out.at[pl.ds(base, BPW)])


def kernel(emb, cs, nodes):
    return _sg_hs_kernel(emb, cs, nodes)
